# baseline (speedup 1.0000x reference)
import jax
import jax.numpy as jnp
from jax.experimental import pallas as pl
from jax.experimental.pallas import tpu as pltpu

B, T, C = 4, 4096, 2048
E = 16
TOP_K = 2
BM = 512


def _router_block(x_ref, w_ref, out_ref, idx_ref):
    logits = jnp.dot(x_ref[...], w_ref[...], preferred_element_type=jnp.float32)
    iota = jax.lax.broadcasted_iota(jnp.int32, (BM, E), 1)

    m1 = jnp.max(logits, axis=-1, keepdims=True)
    i1 = jnp.min(jnp.where(logits == m1, iota, E), axis=-1, keepdims=True)
    masked = jnp.where(iota == i1, -jnp.inf, logits)
    m2 = jnp.max(masked, axis=-1, keepdims=True)
    i2 = jnp.min(jnp.where(masked == m2, iota, E), axis=-1, keepdims=True)

    keep = (iota == i1) | (iota == i2)
    p = jnp.where(keep, jnp.exp(logits - m1), 0.0)
    out_ref[...] = p * (1.0 / (1.0 + jnp.exp(m2 - m1)))
    idx_ref[...] = jnp.concatenate([i1, i2], axis=-1)


def kernel(x, W_route, W_noise):
    del W_noise
    xf = x.reshape(B * T, C)
    wT = W_route.T
    grid = (B * T // BM,)
    out, idx = pl.pallas_call(
        _router_block,
        grid=grid,
        in_specs=[
            pl.BlockSpec((BM, C), lambda i: (i, 0)),
            pl.BlockSpec((C, E), lambda i: (0, 0)),
        ],
        out_specs=[
            pl.BlockSpec((BM, E), lambda i: (i, 0)),
            pl.BlockSpec((BM, TOP_K), lambda i: (i, 0)),
        ],
        out_shape=[
            jax.ShapeDtypeStruct((B * T, E), jnp.float32),
            jax.ShapeDtypeStruct((B * T, TOP_K), jnp.int32),
        ],
        compiler_params=pltpu.CompilerParams(
            dimension_semantics=("parallel",),
        ),
    )(xf, wT)
    return out.reshape(B, T, E), idx.reshape(B, T, TOP_K)


# speedup vs baseline: 1.1652x; 1.1652x over previous
import jax
import jax.numpy as jnp
from jax.experimental import pallas as pl
from jax.experimental.pallas import tpu as pltpu

B, T, C = 4, 4096, 2048
E = 16
TOP_K = 2
BM = 1024


def _router_block(x_ref, w_ref, out_ref, idx_ref):
    logits = jnp.dot(x_ref[...], w_ref[...], preferred_element_type=jnp.float32)
    iota = jax.lax.broadcasted_iota(jnp.int32, (BM, E), 1)

    m1 = jnp.max(logits, axis=-1, keepdims=True)
    i1 = jnp.min(jnp.where(logits == m1, iota, E), axis=-1, keepdims=True)
    masked = jnp.where(iota == i1, -jnp.inf, logits)
    m2 = jnp.max(masked, axis=-1, keepdims=True)
    i2 = jnp.min(jnp.where(masked == m2, iota, E), axis=-1, keepdims=True)

    keep = (iota == i1) | (iota == i2)
    p = jnp.where(keep, jnp.exp(logits - m1), 0.0)
    out_ref[...] = p * (1.0 / (1.0 + jnp.exp(m2 - m1)))
    idx_ref[...] = jnp.concatenate([i1, i2], axis=-1)


def kernel(x, W_route, W_noise):
    del W_noise
    xf = x.reshape(B * T, C)
    wT = W_route.T
    grid = (B * T // BM,)
    out, idx = pl.pallas_call(
        _router_block,
        grid=grid,
        in_specs=[
            pl.BlockSpec((BM, C), lambda i: (i, 0)),
            pl.BlockSpec((C, E), lambda i: (0, 0)),
        ],
        out_specs=[
            pl.BlockSpec((BM, E), lambda i: (i, 0)),
            pl.BlockSpec((BM, TOP_K), lambda i: (i, 0)),
        ],
        out_shape=[
            jax.ShapeDtypeStruct((B * T, E), jnp.float32),
            jax.ShapeDtypeStruct((B * T, TOP_K), jnp.int32),
        ],
        compiler_params=pltpu.CompilerParams(
            dimension_semantics=("parallel",),
        ),
    )(xf, wT)
    return out.reshape(B, T, E), idx.reshape(B, T, TOP_K)


# speedup vs baseline: 1.6991x; 1.4582x over previous
import jax
import jax.numpy as jnp
from jax.experimental import pallas as pl
from jax.experimental.pallas import tpu as pltpu

B, T, C = 4, 4096, 2048
E = 16
TOP_K = 2
BM = 1024
IDX_ROWS = 8


def _router_block(x_ref, w_ref, out_ref, idx_ref):
    logits = jax.lax.dot_general(
        w_ref[...], x_ref[...],
        dimension_numbers=(((1,), (1,)), ((), ())),
        preferred_element_type=jnp.float32,
    )
    iota = jax.lax.broadcasted_iota(jnp.int32, (E, BM), 0)

    m1 = jnp.max(logits, axis=0, keepdims=True)
    i1 = jnp.min(jnp.where(logits == m1, iota, E), axis=0, keepdims=True)
    masked = jnp.where(iota == i1, -jnp.inf, logits)
    m2 = jnp.max(masked, axis=0, keepdims=True)
    i2 = jnp.min(jnp.where(masked == m2, iota, E), axis=0, keepdims=True)

    keep = (iota == i1) | (iota == i2)
    p = jnp.where(keep, jnp.exp(logits - m1), 0.0)
    out_ref[...] = p * (1.0 / (1.0 + jnp.exp(m2 - m1)))
    pair = jnp.concatenate([i1, i2], axis=0)
    idx_ref[...] = jnp.concatenate([pair, pair, pair, pair], axis=0)


def kernel(x, W_route, W_noise):
    del W_noise
    xf = x.reshape(B * T, C)
    grid = (B * T // BM,)
    outT, idxT = pl.pallas_call(
        _router_block,
        grid=grid,
        in_specs=[
            pl.BlockSpec((BM, C), lambda i: (i, 0)),
            pl.BlockSpec((E, C), lambda i: (0, 0)),
        ],
        out_specs=[
            pl.BlockSpec((E, BM), lambda i: (0, i)),
            pl.BlockSpec((IDX_ROWS, BM), lambda i: (0, i)),
        ],
        out_shape=[
            jax.ShapeDtypeStruct((E, B * T), jnp.float32),
            jax.ShapeDtypeStruct((IDX_ROWS, B * T), jnp.int32),
        ],
        compiler_params=pltpu.CompilerParams(
            dimension_semantics=("parallel",),
        ),
    )(xf, W_route)
    router = outT.T.reshape(B, T, E)
    indices = idxT[:TOP_K].T.reshape(B, T, TOP_K)
    return router, indices
